# Initial kernel scaffold; baseline (speedup 1.0000x reference)
import jax
import jax.numpy as jnp
from jax.experimental import pallas as pl


def kernel(x, table):
    raise NotImplementedError("write your pallas kernel here")


# speedup vs baseline: 5.1173x; 5.1173x over previous
import functools

import jax
import jax.numpy as jnp
from jax import lax
from jax.experimental import pallas as pl
from jax.experimental.pallas import tpu as pltpu
from jax.experimental.pallas import tpu_sc as plsc

_NUM_CORES = 2
_NUM_SUBCORES = 16
_NUM_WORKERS = _NUM_CORES * _NUM_SUBCORES
_LANES = 16
_UNROLL = 4


@functools.lru_cache(maxsize=None)
def _make_sc_lookup(n, rows, d, ch):
    per_w = n // _NUM_WORKERS
    steps = per_w // ch
    groups = ch // (_LANES * _UNROLL)
    mesh = plsc.VectorSubcoreMesh(core_axis_name="c", subcore_axis_name="s")

    @functools.partial(
        pl.kernel,
        mesh=mesh,
        out_type=jax.ShapeDtypeStruct((n * d,), jnp.float32),
        compiler_params=pltpu.CompilerParams(needs_layout_passes=False),
        scratch_types=[
            pltpu.VMEM((ch,), jnp.int32),
            pltpu.VMEM((ch * d,), jnp.float32),
            pltpu.VMEM((rows * d,), jnp.float32),
        ],
    )
    def lookup(x_hbm, table_hbm, out_hbm, idx_v, rows_v, tbl_v):
        wid = lax.axis_index("s") * _NUM_CORES + lax.axis_index("c")
        base = wid * per_w
        pltpu.sync_copy(table_hbm, tbl_v)
        lane = lax.iota(jnp.int32, 16)

        def chunk_body(t, carry):
            off = base + t * ch
            pltpu.sync_copy(x_hbm.at[pl.ds(off, ch)], idx_v)

            def group_body(g, c2):
                for u in range(_UNROLL):
                    r0 = g * (_LANES * _UNROLL) + u * _LANES
                    v = idx_v[pl.ds(r0, _LANES)]
                    gidx = v * d
                    sidx = (lane + r0) * d
                    for k in range(d):
                        ck = plsc.load_gather(tbl_v, [gidx + k])
                        plsc.store_scatter(rows_v, [sidx + k], ck)
                return c2

            lax.fori_loop(0, groups, group_body, 0)
            pltpu.sync_copy(rows_v, out_hbm.at[pl.ds(off * d, ch * d)])
            return carry

        lax.fori_loop(0, steps, chunk_body, 0)

    return lookup


def kernel(x, table):
    n = x.size
    rows, d = table.shape
    out = _make_sc_lookup(n, rows, d, 6400)(
        x.reshape(n).astype(jnp.int32), table.reshape(rows * d))
    return out.reshape(x.shape + (d,))


# speedup vs baseline: 5.4655x; 1.0680x over previous
import functools

import jax
import jax.numpy as jnp
from jax import lax
from jax.experimental import pallas as pl
from jax.experimental.pallas import tpu as pltpu
from jax.experimental.pallas import tpu_sc as plsc

_NUM_CORES = 2
_NUM_SUBCORES = 16
_NUM_WORKERS = _NUM_CORES * _NUM_SUBCORES
_LANES = 16
_PIPE = 8


@functools.lru_cache(maxsize=None)
def _make_sc_lookup(n, rows, d, ch):
    per_w = n // _NUM_WORKERS
    steps = per_w // ch
    mesh = plsc.VectorSubcoreMesh(core_axis_name="c", subcore_axis_name="s")

    @functools.partial(
        pl.kernel,
        mesh=mesh,
        out_type=jax.ShapeDtypeStruct((n * d,), jnp.float32),
        compiler_params=pltpu.CompilerParams(needs_layout_passes=False),
        scratch_types=[
            pltpu.VMEM((ch,), jnp.int32),
            pltpu.VMEM((ch,), jnp.int32),
            pltpu.VMEM((ch * d,), jnp.float32),
            pltpu.VMEM((ch * d,), jnp.float32),
            pltpu.VMEM((rows * d,), jnp.float32),
            pltpu.SemaphoreType.DMA,
            pltpu.SemaphoreType.DMA,
            pltpu.SemaphoreType.DMA,
            pltpu.SemaphoreType.DMA,
        ],
    )
    def lookup(x_hbm, table_hbm, out_hbm, idx0, idx1, rows0, rows1, tbl_v,
               si0, si1, so0, so1):
        wid = lax.axis_index("s") * _NUM_CORES + lax.axis_index("c")
        base = wid * per_w
        pltpu.sync_copy(table_hbm, tbl_v)
        lane_d = lax.iota(jnp.int32, _LANES) * d
        idx_bufs, row_bufs = [idx0, idx1], [rows0, rows1]
        in_sems, out_sems = [si0, si1], [so0, so1]

        def start_in(t):
            off = base + t * ch
            return pltpu.async_copy(
                x_hbm.at[pl.ds(off, ch)], idx_bufs[t % 2], in_sems[t % 2])

        def start_out(t):
            off = (base + t * ch) * d
            return pltpu.async_copy(
                row_bufs[t % 2], out_hbm.at[pl.ds(off, ch * d)], out_sems[t % 2])

        in_cp = [None] * steps
        out_cp = [None] * steps
        in_cp[0] = start_in(0)
        for t in range(steps):
            if t + 1 < steps:
                in_cp[t + 1] = start_in(t + 1)
            in_cp[t].wait()
            if t >= 2:
                out_cp[t - 2].wait()
            idx_v = idx_bufs[t % 2]
            rows_v = row_bufs[t % 2]

            @plsc.parallel_loop(0, ch, _LANES, unroll=_PIPE)
            def _body(r0):
                v = idx_v[pl.ds(r0, _LANES)]
                gidx = v * d
                sidx = lane_d + r0 * d
                for k in range(d):
                    ck = plsc.load_gather(tbl_v, [gidx + k])
                    plsc.store_scatter(rows_v, [sidx + k], ck)

            out_cp[t] = start_out(t)
        for t in range(max(steps - 2, 0), steps):
            out_cp[t].wait()

    return lookup


def kernel(x, table):
    n = x.size
    rows, d = table.shape
    out = _make_sc_lookup(n, rows, d, 6400)(
        x.reshape(n).astype(jnp.int32), table.reshape(rows * d))
    return out.reshape(x.shape + (d,))


# speedup vs baseline: 183.3562x; 33.5478x over previous
import functools

import jax
import jax.numpy as jnp
from jax import lax
from jax.experimental import pallas as pl
from jax.experimental.pallas import tpu as pltpu
from jax.experimental.pallas import tpu_sc as plsc

_NUM_CORES = 2
_NUM_SUBCORES = 16
_NUM_WORKERS = _NUM_CORES * _NUM_SUBCORES
_LANES = 16
_JB = 8


@functools.lru_cache(maxsize=None)
def _make_sc_lookup(nj, ni, rows, d):
    w = ni // _NUM_WORKERS
    steps = nj // _JB
    units = (_JB * w) // _LANES
    cg = w // _LANES
    mesh = plsc.VectorSubcoreMesh(core_axis_name="c", subcore_axis_name="s")

    @functools.partial(
        pl.kernel,
        mesh=mesh,
        out_type=jax.ShapeDtypeStruct((nj, d, ni), jnp.float32),
        compiler_params=pltpu.CompilerParams(needs_layout_passes=False),
        scratch_types=[
            pltpu.VMEM((_JB, w), jnp.int32),
            pltpu.VMEM((_JB, w), jnp.int32),
            pltpu.VMEM((_JB, d, w), jnp.float32),
            pltpu.VMEM((_JB, d, w), jnp.float32),
            pltpu.VMEM((rows * d,), jnp.float32),
            pltpu.SemaphoreType.DMA,
            pltpu.SemaphoreType.DMA,
            pltpu.SemaphoreType.DMA,
            pltpu.SemaphoreType.DMA,
        ],
    )
    def lookup(xt_hbm, table_hbm, out_hbm, idx0, idx1, rows0, rows1, tbl_v,
               si0, si1, so0, so1):
        wid = lax.axis_index("s") * _NUM_CORES + lax.axis_index("c")
        i0 = wid * w
        pltpu.sync_copy(table_hbm, tbl_v)
        idx_bufs, row_bufs = [idx0, idx1], [rows0, rows1]
        in_sems, out_sems = [si0, si1], [so0, so1]

        def start_in(t):
            return pltpu.async_copy(
                xt_hbm.at[pl.ds(t * _JB, _JB), pl.ds(i0, w)],
                idx_bufs[t % 2], in_sems[t % 2])

        def start_out(t):
            return pltpu.async_copy(
                row_bufs[t % 2],
                out_hbm.at[pl.ds(t * _JB, _JB), :, pl.ds(i0, w)],
                out_sems[t % 2])

        in_cp = [None] * steps
        out_cp = [None] * steps
        in_cp[0] = start_in(0)
        for t in range(steps):
            if t + 1 < steps:
                in_cp[t + 1] = start_in(t + 1)
            in_cp[t].wait()
            if t >= 2:
                out_cp[t - 2].wait()
            idx_v = idx_bufs[t % 2]
            rows_v = row_bufs[t % 2]

            @plsc.parallel_loop(0, units, 1, unroll=8)
            def _body(u):
                r = u // cg
                c = (u % cg) * _LANES
                v = idx_v[r, pl.ds(c, _LANES)]
                gidx = v * d
                for k in range(d):
                    rows_v[r, k, pl.ds(c, _LANES)] = plsc.load_gather(
                        tbl_v, [gidx + k])

            out_cp[t] = start_out(t)
        for t in range(max(steps - 2, 0), steps):
            out_cp[t].wait()

    return lookup


def kernel(x, table):
    ni, nj = x.shape
    rows, d = table.shape
    out_t = _make_sc_lookup(nj, ni, rows, d)(
        x.T.astype(jnp.int32), table.reshape(rows * d))
    return out_t.transpose(2, 0, 1)


# speedup vs baseline: 237.5062x; 1.2953x over previous
import functools

import jax
import jax.numpy as jnp
from jax import lax
from jax.experimental import pallas as pl
from jax.experimental.pallas import tpu as pltpu
from jax.experimental.pallas import tpu_sc as plsc

_NUM_CORES = 2
_NUM_SUBCORES = 16
_NUM_WORKERS = _NUM_CORES * _NUM_SUBCORES
_LANES = 16
_JB = 8


@functools.lru_cache(maxsize=None)
def _make_sc_lookup(nj, ni, rows, d):
    w = ni // _NUM_WORKERS
    steps = nj // _JB
    units = (_JB * w) // _LANES
    cg = w // _LANES
    mesh = plsc.VectorSubcoreMesh(core_axis_name="c", subcore_axis_name="s")

    @functools.partial(
        pl.kernel,
        mesh=mesh,
        out_type=jax.ShapeDtypeStruct((nj, d, ni), jnp.float32),
        compiler_params=pltpu.CompilerParams(needs_layout_passes=False),
        scratch_types=[
            pltpu.VMEM((_JB, w), jnp.int32),
            pltpu.VMEM((_JB, w), jnp.int32),
            pltpu.VMEM((_JB, d, w), jnp.float32),
            pltpu.VMEM((_JB, d, w), jnp.float32),
            pltpu.VMEM((d, _LANES), jnp.float32),
            pltpu.SemaphoreType.DMA,
            pltpu.SemaphoreType.DMA,
            pltpu.SemaphoreType.DMA,
            pltpu.SemaphoreType.DMA,
        ],
    )
    def lookup(xt_hbm, table_hbm, out_hbm, idx0, idx1, rows0, rows1, tbl_v,
               si0, si1, so0, so1):
        wid = lax.axis_index("s") * _NUM_CORES + lax.axis_index("c")
        i0 = wid * w
        pltpu.sync_copy(table_hbm, tbl_v)
        idx_bufs, row_bufs = [idx0, idx1], [rows0, rows1]
        in_sems, out_sems = [si0, si1], [so0, so1]

        def start_in(t):
            return pltpu.async_copy(
                xt_hbm.at[pl.ds(t * _JB, _JB), pl.ds(i0, w)],
                idx_bufs[t % 2], in_sems[t % 2])

        def start_out(t):
            return pltpu.async_copy(
                row_bufs[t % 2],
                out_hbm.at[pl.ds(t * _JB, _JB), :, pl.ds(i0, w)],
                out_sems[t % 2])

        in_cp = [None] * steps
        out_cp = [None] * steps
        in_cp[0] = start_in(0)
        for t in range(steps):
            if t + 1 < steps:
                in_cp[t + 1] = start_in(t + 1)
            in_cp[t].wait()
            if t >= 2:
                out_cp[t - 2].wait()
            idx_v = idx_bufs[t % 2]
            rows_v = row_bufs[t % 2]

            @plsc.parallel_loop(0, units, 1, unroll=8)
            def _body(u):
                r = u // cg
                c = (u % cg) * _LANES
                v = idx_v[r, pl.ds(c, _LANES)]
                for k in range(d):
                    rows_v[r, k, pl.ds(c, _LANES)] = plsc.load_gather(
                        tbl_v.at[k], [v])

            out_cp[t] = start_out(t)
        for t in range(max(steps - 2, 0), steps):
            out_cp[t].wait()

    return lookup


def kernel(x, table):
    ni, nj = x.shape
    rows, d = table.shape
    tcols = jnp.zeros((d, _LANES), table.dtype).at[:, :rows].set(table.T)
    out_t = _make_sc_lookup(nj, ni, rows, d)(x.T.astype(jnp.int32), tcols)
    return out_t.transpose(2, 0, 1)
